# Initial kernel scaffold; baseline (speedup 1.0000x reference)
import jax
import jax.numpy as jnp
from jax.experimental import pallas as pl


def kernel(token_ids, weight, proj):
    raise NotImplementedError("write your pallas kernel here")


# speedup vs baseline: 3.8548x; 3.8548x over previous
import functools

import jax
import jax.numpy as jnp
from jax import lax
from jax.experimental import pallas as pl
from jax.experimental.pallas import tpu as pltpu
from jax.experimental.pallas import tpu_sc as plsc

_NUM_BUCKETS = 100000
_RANK = 32
_MODEL_DIM = 128

_NC = 2
_NS = 16
_NW = _NC * _NS
_L = 16

_CHUNK = 128


def _gather_sum_body(ntok_per_w, nchunks, tok_hbm, w_hbm, out_hbm,
                     tok_v, i1_v, i2_v, r1_v, r2_v, s1, s2):
    wid = lax.axis_index("s") * _NC + lax.axis_index("c")
    base = wid * ntok_per_w

    @pl.loop(0, nchunks)
    def _chunk(ci):
        off = base + ci * _CHUNK
        pltpu.sync_copy(tok_hbm.at[pl.ds(off, _CHUNK)], tok_v)

        @pl.loop(0, _CHUNK // _L)
        def _hash(i):
            t = tok_v[pl.ds(i * _L, _L)]
            i1_v[pl.ds(i * _L, _L)] = (t * 31 + 7) % _NUM_BUCKETS
            i2_v[pl.ds(i * _L, _L)] = (t * 131 + 13) % _NUM_BUCKETS

        cp1 = pltpu.async_copy(w_hbm.at[i1_v], r1_v, s1)
        cp2 = pltpu.async_copy(w_hbm.at[i2_v], r2_v, s2)
        cp1.wait()
        cp2.wait()

        @pl.loop(0, _CHUNK)
        def _add(j):
            r1_v[j, pl.ds(0, _L)] = r1_v[j, pl.ds(0, _L)] + r2_v[j, pl.ds(0, _L)]
            r1_v[j, pl.ds(_L, _L)] = r1_v[j, pl.ds(_L, _L)] + r2_v[j, pl.ds(_L, _L)]

        pltpu.sync_copy(r1_v, out_hbm.at[pl.ds(off, _CHUNK)])


def _sc_gather_sum(tok_flat):
    n = tok_flat.shape[0]
    ntok_per_w = n // _NW
    nchunks = ntok_per_w // _CHUNK
    mesh = plsc.VectorSubcoreMesh(core_axis_name="c", subcore_axis_name="s")
    body = functools.partial(_gather_sum_body, ntok_per_w, nchunks)
    return pl.kernel(
        body,
        out_type=jax.ShapeDtypeStruct((n, _RANK), jnp.float32),
        mesh=mesh,
        scratch_types=[
            pltpu.VMEM((_CHUNK,), jnp.int32),
            pltpu.VMEM((_CHUNK,), jnp.int32),
            pltpu.VMEM((_CHUNK,), jnp.int32),
            pltpu.VMEM((_CHUNK, _RANK), jnp.float32),
            pltpu.VMEM((_CHUNK, _RANK), jnp.float32),
            pltpu.SemaphoreType.DMA,
            pltpu.SemaphoreType.DMA,
        ],
        compiler_params=pltpu.CompilerParams(use_tc_tiling_on_sc=False),
    )


def _proj_body(x_ref, p_ref, o_ref):
    o_ref[...] = jnp.dot(x_ref[...], p_ref[...],
                         preferred_element_type=jnp.float32)


def _tc_project(summed, proj_half):
    n = summed.shape[0]
    bt = 4096
    return pl.pallas_call(
        _proj_body,
        grid=(n // bt,),
        in_specs=[
            pl.BlockSpec((bt, _RANK), lambda i: (i, 0)),
            pl.BlockSpec((_RANK, _MODEL_DIM), lambda i: (0, 0)),
        ],
        out_specs=pl.BlockSpec((bt, _MODEL_DIM), lambda i: (i, 0)),
        out_shape=jax.ShapeDtypeStruct((n, _MODEL_DIM), jnp.float32),
    )(summed, proj_half)


def kernel(token_ids, weight, proj):
    b, h = token_ids.shape
    tok_flat = token_ids.reshape(b * h)
    summed = _sc_gather_sum(tok_flat)(tok_flat, weight)
    out = _tc_project(summed, proj * 0.5)
    return out.reshape(b, h, _MODEL_DIM)


# speedup vs baseline: 4.6586x; 1.2085x over previous
import functools

import jax
import jax.numpy as jnp
from jax import lax
from jax.experimental import pallas as pl
from jax.experimental.pallas import tpu as pltpu
from jax.experimental.pallas import tpu_sc as plsc

_NUM_BUCKETS = 100000
_RANK = 32
_MODEL_DIM = 128

_NC = 2
_NS = 16
_NW = _NC * _NS
_L = 16

_CHUNK = 128
_NBUF = 4
_NWBUF = 2


def _gather_sum_body(ntok_per_w, nchunks, tok_hbm, w_hbm, out_hbm,
                     tok_v, i1_v, i2_v, r1_v, r2_v, w_v, s1, s2, so):
    wid = lax.axis_index("s") * _NC + lax.axis_index("c")
    base = wid * ntok_per_w

    pltpu.sync_copy(tok_hbm.at[pl.ds(base, ntok_per_w)], tok_v)

    def start(ci, b):
        @pl.loop(0, _CHUNK // _L)
        def _hash(i):
            t = tok_v[pl.ds(ci * _CHUNK + i * _L, _L)]
            i1_v[b, pl.ds(i * _L, _L)] = (t * 31 + 7) % _NUM_BUCKETS
            i2_v[b, pl.ds(i * _L, _L)] = (t * 131 + 13) % _NUM_BUCKETS

        pltpu.async_copy(w_hbm.at[i1_v.at[b]], r1_v.at[b], s1.at[b])
        pltpu.async_copy(w_hbm.at[i2_v.at[b]], r2_v.at[b], s2.at[b])

    def wait_gathers(b):
        pltpu.make_async_copy(w_hbm.at[i1_v.at[b]], r1_v.at[b], s1.at[b]).wait()
        pltpu.make_async_copy(w_hbm.at[i2_v.at[b]], r2_v.at[b], s2.at[b]).wait()

    def drain_out(ci, w):
        pltpu.make_async_copy(
            w_v.at[w], out_hbm.at[pl.ds(base + ci * _CHUNK, _CHUNK)],
            so.at[w]).wait()

    for b in range(_NBUF - 1):
        start(b, b)

    def process(ci, b, w):
        wait_gathers(b)
        nb = (b - 1) % _NBUF

        @pl.when(ci + _NBUF - 1 < nchunks)
        def _():
            start(ci + _NBUF - 1, nb)

        @pl.when(ci >= _NWBUF)
        def _():
            drain_out(ci - _NWBUF, w)

        @plsc.parallel_loop(0, _CHUNK, 1, unroll=4)
        def _add(j):
            w_v[w, j, pl.ds(0, _L)] = (r1_v[b, j, pl.ds(0, _L)]
                                       + r2_v[b, j, pl.ds(0, _L)])
            w_v[w, j, pl.ds(_L, _L)] = (r1_v[b, j, pl.ds(_L, _L)]
                                        + r2_v[b, j, pl.ds(_L, _L)])

        pltpu.async_copy(
            w_v.at[w], out_hbm.at[pl.ds(base + ci * _CHUNK, _CHUNK)],
            so.at[w])

    @pl.loop(0, nchunks // _NBUF)
    def _group(g):
        for b in range(_NBUF):
            ci = g * _NBUF + b
            process(ci, b, b % _NWBUF)

    drain_out(nchunks - 2, (nchunks - 2) % _NWBUF)
    drain_out(nchunks - 1, (nchunks - 1) % _NWBUF)


def _sc_gather_sum(tok_flat):
    n = tok_flat.shape[0]
    ntok_per_w = n // _NW
    nchunks = ntok_per_w // _CHUNK
    assert nchunks % _NBUF == 0 and nchunks >= 2 * _NBUF
    mesh = plsc.VectorSubcoreMesh(core_axis_name="c", subcore_axis_name="s")
    body = functools.partial(_gather_sum_body, ntok_per_w, nchunks)
    return pl.kernel(
        body,
        out_type=jax.ShapeDtypeStruct((n, _RANK), jnp.float32),
        mesh=mesh,
        scratch_types=[
            pltpu.VMEM((ntok_per_w,), jnp.int32),
            pltpu.VMEM((_NBUF, _CHUNK), jnp.int32),
            pltpu.VMEM((_NBUF, _CHUNK), jnp.int32),
            pltpu.VMEM((_NBUF, _CHUNK, _RANK), jnp.float32),
            pltpu.VMEM((_NBUF, _CHUNK, _RANK), jnp.float32),
            pltpu.VMEM((_NWBUF, _CHUNK, _RANK), jnp.float32),
            pltpu.SemaphoreType.DMA((_NBUF,)),
            pltpu.SemaphoreType.DMA((_NBUF,)),
            pltpu.SemaphoreType.DMA((_NWBUF,)),
        ],
        compiler_params=pltpu.CompilerParams(use_tc_tiling_on_sc=False),
    )


def _proj_body(x_ref, p_ref, o_ref):
    o_ref[...] = jnp.dot(x_ref[...], p_ref[...],
                         preferred_element_type=jnp.float32)


def _tc_project(summed, proj_half):
    n = summed.shape[0]
    bt = 4096
    return pl.pallas_call(
        _proj_body,
        grid=(n // bt,),
        in_specs=[
            pl.BlockSpec((bt, _RANK), lambda i: (i, 0)),
            pl.BlockSpec((_RANK, _MODEL_DIM), lambda i: (0, 0)),
        ],
        out_specs=pl.BlockSpec((bt, _MODEL_DIM), lambda i: (i, 0)),
        out_shape=jax.ShapeDtypeStruct((n, _MODEL_DIM), jnp.float32),
    )(summed, proj_half)


def kernel(token_ids, weight, proj):
    b, h = token_ids.shape
    tok_flat = token_ids.reshape(b * h)
    summed = _sc_gather_sum(tok_flat)(tok_flat, weight)
    out = _tc_project(summed, proj * 0.5)
    return out.reshape(b, h, _MODEL_DIM)


# speedup vs baseline: 5.8287x; 1.2512x over previous
import functools

import jax
import jax.numpy as jnp
from jax import lax
from jax.experimental import pallas as pl
from jax.experimental.pallas import tpu as pltpu
from jax.experimental.pallas import tpu_sc as plsc

_NUM_BUCKETS = 100000
_RANK = 32
_MODEL_DIM = 128

_NC = 2
_NS = 16
_NW = _NC * _NS
_L = 16

_CHUNK = 128
_NBUF = 4
_NWBUF = 2
_ROWS = _CHUNK // 4


def _gather_sum_body(ntok_per_w, nchunks, tok_hbm, w_hbm, out_hbm,
                     tok_v, i1_v, i2_v, r1_v, r2_v, w_v, s1, s2, so):
    wid = lax.axis_index("s") * _NC + lax.axis_index("c")
    base = wid * ntok_per_w
    row_base = base // 4

    pltpu.sync_copy(tok_hbm.at[pl.ds(base, ntok_per_w)], tok_v)

    def start(ci, b):
        @pl.loop(0, _CHUNK // _L)
        def _hash(i):
            t = tok_v[pl.ds(ci * _CHUNK + i * _L, _L)]
            i1_v[b, pl.ds(i * _L, _L)] = (t * 31 + 7) % _NUM_BUCKETS
            i2_v[b, pl.ds(i * _L, _L)] = (t * 131 + 13) % _NUM_BUCKETS

        pltpu.async_copy(w_hbm.at[i1_v.at[b]], r1_v.at[b], s1.at[b])
        pltpu.async_copy(w_hbm.at[i2_v.at[b]], r2_v.at[b], s2.at[b])

    def wait_gathers(b):
        pltpu.make_async_copy(w_hbm.at[i1_v.at[b]], r1_v.at[b], s1.at[b]).wait()
        pltpu.make_async_copy(w_hbm.at[i2_v.at[b]], r2_v.at[b], s2.at[b]).wait()

    def drain_out(ci, w):
        pltpu.make_async_copy(
            w_v.at[w], out_hbm.at[pl.ds(row_base + ci * _ROWS, _ROWS)],
            so.at[w]).wait()

    for b in range(_NBUF - 1):
        start(b, b)

    def process(ci, b, w):
        wait_gathers(b)
        nb = (b - 1) % _NBUF

        @pl.when(ci + _NBUF - 1 < nchunks)
        def _():
            start(ci + _NBUF - 1, nb)

        @pl.when(ci >= _NWBUF)
        def _():
            drain_out(ci - _NWBUF, w)

        @plsc.parallel_loop(0, _ROWS, 1, unroll=2)
        def _add(jr):
            for q in range(4):
                j = jr * 4 + q
                for h in range(2):
                    w_v[w, jr, pl.ds(q * 32 + h * _L, _L)] = (
                        r1_v[b, j, pl.ds(h * _L, _L)]
                        + r2_v[b, j, pl.ds(h * _L, _L)])

        pltpu.async_copy(
            w_v.at[w], out_hbm.at[pl.ds(row_base + ci * _ROWS, _ROWS)],
            so.at[w])

    @pl.loop(0, nchunks // _NBUF)
    def _group(g):
        for b in range(_NBUF):
            process(g * _NBUF + b, b, b % _NWBUF)

    drain_out(nchunks - 2, (nchunks - 2) % _NWBUF)
    drain_out(nchunks - 1, (nchunks - 1) % _NWBUF)


def _sc_gather_sum(tok_flat):
    n = tok_flat.shape[0]
    ntok_per_w = n // _NW
    nchunks = ntok_per_w // _CHUNK
    assert nchunks % _NBUF == 0 and nchunks >= 2 * _NBUF
    mesh = plsc.VectorSubcoreMesh(core_axis_name="c", subcore_axis_name="s")
    body = functools.partial(_gather_sum_body, ntok_per_w, nchunks)
    return pl.kernel(
        body,
        out_type=jax.ShapeDtypeStruct((n // 4, 128), jnp.float32),
        mesh=mesh,
        scratch_types=[
            pltpu.VMEM((ntok_per_w,), jnp.int32),
            pltpu.VMEM((_NBUF, _CHUNK), jnp.int32),
            pltpu.VMEM((_NBUF, _CHUNK), jnp.int32),
            pltpu.VMEM((_NBUF, _CHUNK, _RANK), jnp.float32),
            pltpu.VMEM((_NBUF, _CHUNK, _RANK), jnp.float32),
            pltpu.VMEM((_NWBUF, _ROWS, 128), jnp.float32),
            pltpu.SemaphoreType.DMA((_NBUF,)),
            pltpu.SemaphoreType.DMA((_NBUF,)),
            pltpu.SemaphoreType.DMA((_NWBUF,)),
        ],
        compiler_params=pltpu.CompilerParams(use_tc_tiling_on_sc=False),
    )


def _proj_body(bb, hist, x_ref, p_ref, o_ref):
    x = x_ref[...]
    y = jnp.dot(x, p_ref[...], preferred_element_type=jnp.float32)
    y = y.reshape(bb * hist, _MODEL_DIM)
    for b in range(bb):
        o_ref[b] = y[b * hist:(b + 1) * hist]


def _tc_project(x2d, proj_bd, batch, hist):
    bb = 16
    grid = batch // bb
    rows_per_block = bb * hist // 4
    return pl.pallas_call(
        functools.partial(_proj_body, bb, hist),
        grid=(grid,),
        in_specs=[
            pl.BlockSpec((rows_per_block, 128), lambda i: (i, 0)),
            pl.BlockSpec((_MODEL_DIM, 4 * _MODEL_DIM), lambda i: (0, 0)),
        ],
        out_specs=pl.BlockSpec((bb, hist, _MODEL_DIM), lambda i: (i, 0, 0)),
        out_shape=jax.ShapeDtypeStruct((batch, hist, _MODEL_DIM),
                                       jnp.float32),
    )(x2d, proj_bd)


def kernel(token_ids, weight, proj):
    b, h = token_ids.shape
    tok_flat = token_ids.reshape(b * h)
    x2d = _sc_gather_sum(tok_flat)(tok_flat, weight)
    ph = proj * 0.5
    proj_bd = jnp.zeros((_MODEL_DIM, 4 * _MODEL_DIM), jnp.float32)
    for q in range(4):
        proj_bd = proj_bd.at[q * _RANK:(q + 1) * _RANK,
                             q * _MODEL_DIM:(q + 1) * _MODEL_DIM].set(ph)
    return _tc_project(x2d, proj_bd, b, h)


# speedup vs baseline: 11.4020x; 1.9562x over previous
import functools

import jax
import jax.numpy as jnp
from jax import lax
from jax.experimental import pallas as pl
from jax.experimental.pallas import tpu as pltpu
from jax.experimental.pallas import tpu_sc as plsc

_NUM_BUCKETS = 100000
_RANK = 32
_MODEL_DIM = 128

_NC = 2
_NS = 16
_NW = _NC * _NS
_L = 16

_CHUNK = 256
_GT = 128
_NBUF = 4
_NWBUF = 2
_ROWS = _CHUNK // 4
_KCH = 8


def _gather_sum_body(ntok_per_w, nchunks, tok_hbm, w_hbm, out_hbm,
                     tok_v, i1_v, i2_v, r1_v, r2_v, w_v, s1, s2, so):
    wid = lax.axis_index("s") * _NC + lax.axis_index("c")
    base = wid * ntok_per_w
    row_base = base // 4

    def start(ci, b):
        pltpu.sync_copy(tok_hbm.at[pl.ds(base + ci * _CHUNK, _CHUNK)],
                        tok_v.at[b])

        @pl.loop(0, _CHUNK // _L)
        def _hash(i):
            t = tok_v[b, pl.ds(i * _L, _L)]
            i1_v[b, pl.ds(i * _L, _L)] = (t * 31 + 7) % _NUM_BUCKETS
            i2_v[b, pl.ds(i * _L, _L)] = (t * 131 + 13) % _NUM_BUCKETS

        for g in range(_CHUNK // _GT):
            pltpu.async_copy(w_hbm.at[i1_v.at[b, pl.ds(g * _GT, _GT)]],
                             r1_v.at[b, pl.ds(g * _GT, _GT)], s1.at[b])
            pltpu.async_copy(w_hbm.at[i2_v.at[b, pl.ds(g * _GT, _GT)]],
                             r2_v.at[b, pl.ds(g * _GT, _GT)], s2.at[b])

    def wait_gathers(b):
        for g in range(_CHUNK // _GT):
            pltpu.make_async_copy(
                w_hbm.at[i1_v.at[b, pl.ds(g * _GT, _GT)]],
                r1_v.at[b, pl.ds(g * _GT, _GT)], s1.at[b]).wait()
            pltpu.make_async_copy(
                w_hbm.at[i2_v.at[b, pl.ds(g * _GT, _GT)]],
                r2_v.at[b, pl.ds(g * _GT, _GT)], s2.at[b]).wait()

    def drain_out(ci, w):
        pltpu.make_async_copy(
            w_v.at[w], out_hbm.at[pl.ds(row_base + ci * _ROWS, _ROWS)],
            so.at[w]).wait()

    for b in range(_NBUF - 1):
        start(b, b)

    def process(ci, b, w):
        wait_gathers(b)
        nb = (b - 1) % _NBUF

        @pl.when(ci + _NBUF - 1 < nchunks)
        def _():
            start(ci + _NBUF - 1, nb)

        @pl.when(ci >= _NWBUF)
        def _():
            drain_out(ci - _NWBUF, w)

        @plsc.parallel_loop(0, _ROWS, 1, unroll=2)
        def _add(jr):
            for q in range(4):
                j = q * _ROWS + jr
                for h in range(2):
                    w_v[w, jr, pl.ds(q * 32 + h * _L, _L)] = (
                        r1_v[b, j, pl.ds(h * _L, _L)]
                        + r2_v[b, j, pl.ds(h * _L, _L)])

        pltpu.async_copy(
            w_v.at[w], out_hbm.at[pl.ds(row_base + ci * _ROWS, _ROWS)],
            so.at[w])

    @pl.loop(0, nchunks // _NBUF)
    def _group(g):
        for k in range(_NBUF):
            ci = g * _NBUF + k
            process(ci, k, k % _NWBUF)

    drain_out(nchunks - 2, (nchunks - 2) % _NWBUF)
    drain_out(nchunks - 1, (nchunks - 1) % _NWBUF)


def _sc_gather_sum(tok_hm):
    n = tok_hm.shape[0]
    ntok_per_w = n // _NW
    nchunks = ntok_per_w // _CHUNK
    assert nchunks % _NBUF == 0 and nchunks >= 2 * _NBUF
    mesh = plsc.VectorSubcoreMesh(core_axis_name="c", subcore_axis_name="s")
    body = functools.partial(_gather_sum_body, ntok_per_w, nchunks)
    return pl.kernel(
        body,
        out_type=jax.ShapeDtypeStruct((n // 4, 128), jnp.float32),
        mesh=mesh,
        scratch_types=[
            pltpu.VMEM((_NBUF, _CHUNK), jnp.int32),
            pltpu.VMEM((_NBUF, _CHUNK), jnp.int32),
            pltpu.VMEM((_NBUF, _CHUNK), jnp.int32),
            pltpu.VMEM((_NBUF, _CHUNK, _RANK), jnp.float32),
            pltpu.VMEM((_NBUF, _CHUNK, _RANK), jnp.float32),
            pltpu.VMEM((_NWBUF, _ROWS, 128), jnp.float32),
            pltpu.SemaphoreType.DMA((_NBUF,)),
            pltpu.SemaphoreType.DMA((_NBUF,)),
            pltpu.SemaphoreType.DMA((_NWBUF,)),
        ],
        compiler_params=pltpu.CompilerParams(use_tc_tiling_on_sc=False),
    )


def _proj_body(x_ref, p_ref, o_ref):
    y = jnp.dot(x_ref[...], p_ref[...], preferred_element_type=jnp.float32)
    for c in range(_KCH):
        for q in range(4):
            o_ref[0, pl.ds(c * _CHUNK + q * _ROWS, _ROWS), :] = (
                y[c * _ROWS:(c + 1) * _ROWS,
                  q * _MODEL_DIM:(q + 1) * _MODEL_DIM])


def _tc_project(x2d, proj_bd, batch, hist):
    tok_per_block = _KCH * _CHUNK
    blocks_per_h = batch // tok_per_block
    grid = hist * blocks_per_h
    return pl.pallas_call(
        _proj_body,
        grid=(grid,),
        in_specs=[
            pl.BlockSpec((tok_per_block // 4, 128), lambda i: (i, 0)),
            pl.BlockSpec((_MODEL_DIM, 4 * _MODEL_DIM), lambda i: (0, 0)),
        ],
        out_specs=pl.BlockSpec(
            (1, tok_per_block, _MODEL_DIM),
            lambda i: (i // blocks_per_h, i % blocks_per_h, 0)),
        out_shape=jax.ShapeDtypeStruct((hist, batch, _MODEL_DIM),
                                       jnp.float32),
    )(x2d, proj_bd)


def kernel(token_ids, weight, proj):
    b, h = token_ids.shape
    tok_hm = jnp.transpose(token_ids).reshape(b * h)
    x2d = _sc_gather_sum(tok_hm)(tok_hm, weight)
    ph = proj * 0.5
    proj_bd = jnp.zeros((_MODEL_DIM, 4 * _MODEL_DIM), jnp.float32)
    for q in range(4):
        proj_bd = proj_bd.at[q * _RANK:(q + 1) * _RANK,
                             q * _MODEL_DIM:(q + 1) * _MODEL_DIM].set(ph)
    out_t = _tc_project(x2d, proj_bd, b, h)
    return jnp.transpose(out_t, (1, 0, 2))


# speedup vs baseline: 13.8786x; 1.2172x over previous
import functools

import jax
import jax.numpy as jnp
from jax import lax
from jax.experimental import pallas as pl
from jax.experimental.pallas import tpu as pltpu
from jax.experimental.pallas import tpu_sc as plsc

_NUM_BUCKETS = 100000
_RANK = 32
_MODEL_DIM = 128

_NC = 2
_NS = 16
_NW = _NC * _NS
_L = 16

_CHUNK = 256
_GT = 128
_NBUF = 4
_NWBUF = 2
_ROWS = _CHUNK // 4
_KCH = 8


def _gather_sum_body(ntok_per_w, nchunks, tok_hbm, w_hbm, out_hbm,
                     tok_v, i1_v, i2_v, r1_v, r2_v, w_v, s1, s2, so):
    wid = lax.axis_index("s") * _NC + lax.axis_index("c")
    base = wid * ntok_per_w
    row_base = base // 4

    def start(ci, b):
        pltpu.sync_copy(tok_hbm.at[pl.ds(base + ci * _CHUNK, _CHUNK)],
                        tok_v.at[b])

        @pl.loop(0, _CHUNK // _L)
        def _hash(i):
            t = tok_v[b, pl.ds(i * _L, _L)]
            i1_v[b, pl.ds(i * _L, _L)] = (t * 31 + 7) % _NUM_BUCKETS
            i2_v[b, pl.ds(i * _L, _L)] = (t * 131 + 13) % _NUM_BUCKETS

        for g in range(_CHUNK // _GT):
            pltpu.async_copy(w_hbm.at[i1_v.at[b, pl.ds(g * _GT, _GT)]],
                             r1_v.at[b, pl.ds(g * _GT, _GT)], s1.at[b])
            pltpu.async_copy(w_hbm.at[i2_v.at[b, pl.ds(g * _GT, _GT)]],
                             r2_v.at[b, pl.ds(g * _GT, _GT)], s2.at[b])

    def wait_gathers(b):
        for g in range(_CHUNK // _GT):
            pltpu.make_async_copy(
                w_hbm.at[i1_v.at[b, pl.ds(g * _GT, _GT)]],
                r1_v.at[b, pl.ds(g * _GT, _GT)], s1.at[b]).wait()
            pltpu.make_async_copy(
                w_hbm.at[i2_v.at[b, pl.ds(g * _GT, _GT)]],
                r2_v.at[b, pl.ds(g * _GT, _GT)], s2.at[b]).wait()

    def drain_out(ci, w):
        pltpu.make_async_copy(
            w_v.at[w], out_hbm.at[pl.ds(row_base + ci * _ROWS, _ROWS)],
            so.at[w]).wait()

    for b in range(_NBUF - 1):
        start(b, b)

    def process(ci, b, w):
        wait_gathers(b)
        nb = (b - 1) % _NBUF

        @pl.when(ci + _NBUF - 1 < nchunks)
        def _():
            start(ci + _NBUF - 1, nb)

        @pl.when(ci >= _NWBUF)
        def _():
            drain_out(ci - _NWBUF, w)

        @plsc.parallel_loop(0, _ROWS, 1, unroll=2)
        def _add(jr):
            for q in range(4):
                j = q * _ROWS + jr
                for h in range(2):
                    w_v[w, jr, pl.ds(q * 32 + h * _L, _L)] = (
                        r1_v[b, j, pl.ds(h * _L, _L)]
                        + r2_v[b, j, pl.ds(h * _L, _L)])

        pltpu.async_copy(
            w_v.at[w], out_hbm.at[pl.ds(row_base + ci * _ROWS, _ROWS)],
            so.at[w])

    @pl.loop(0, nchunks // _NBUF)
    def _group(g):
        for k in range(_NBUF):
            ci = g * _NBUF + k
            process(ci, k, k % _NWBUF)

    drain_out(nchunks - 2, (nchunks - 2) % _NWBUF)
    drain_out(nchunks - 1, (nchunks - 1) % _NWBUF)


def _sc_gather_sum(tok_hm):
    n = tok_hm.shape[0]
    ntok_per_w = n // _NW
    nchunks = ntok_per_w // _CHUNK
    assert nchunks % _NBUF == 0 and nchunks >= 2 * _NBUF
    mesh = plsc.VectorSubcoreMesh(core_axis_name="c", subcore_axis_name="s")
    body = functools.partial(_gather_sum_body, ntok_per_w, nchunks)
    return pl.kernel(
        body,
        out_type=jax.ShapeDtypeStruct((n // 4, 128), jnp.float32),
        mesh=mesh,
        scratch_types=[
            pltpu.VMEM((_NBUF, _CHUNK), jnp.int32),
            pltpu.VMEM((_NBUF, _CHUNK), jnp.int32),
            pltpu.VMEM((_NBUF, _CHUNK), jnp.int32),
            pltpu.VMEM((_NBUF, _CHUNK, _RANK), jnp.float32),
            pltpu.VMEM((_NBUF, _CHUNK, _RANK), jnp.float32),
            pltpu.VMEM((_NWBUF, _ROWS, 128), jnp.float32),
            pltpu.SemaphoreType.DMA((_NBUF,)),
            pltpu.SemaphoreType.DMA((_NBUF,)),
            pltpu.SemaphoreType.DMA((_NWBUF,)),
        ],
        compiler_params=pltpu.CompilerParams(use_tc_tiling_on_sc=False),
    )


def _proj_body(x_ref, p_ref, o_ref):
    y = jnp.dot(x_ref[...], p_ref[...], preferred_element_type=jnp.float32)
    for c in range(_KCH):
        for q in range(4):
            o_ref[0, pl.ds(c * _CHUNK + q * _ROWS, _ROWS), :] = (
                y[c * _ROWS:(c + 1) * _ROWS,
                  q * _MODEL_DIM:(q + 1) * _MODEL_DIM])


def _proj_body_alias(x_ref, p_ref, prev_ref, o_ref):
    del prev_ref
    _proj_body(x_ref, p_ref, o_ref)


def _tc_project_first(x2d, proj_bd, batch, hist, hist_lo):
    tok_per_block = _KCH * _CHUNK
    blocks_per_h = batch // tok_per_block
    grid = hist_lo * blocks_per_h
    return pl.pallas_call(
        _proj_body,
        grid=(grid,),
        in_specs=[
            pl.BlockSpec((tok_per_block // 4, 128), lambda i: (i, 0)),
            pl.BlockSpec((_MODEL_DIM, 4 * _MODEL_DIM), lambda i: (0, 0)),
        ],
        out_specs=pl.BlockSpec(
            (1, tok_per_block, _MODEL_DIM),
            lambda i: (i // blocks_per_h, i % blocks_per_h, 0)),
        out_shape=jax.ShapeDtypeStruct((hist, batch, _MODEL_DIM),
                                       jnp.float32),
    )(x2d, proj_bd)


def _tc_project_second(x2d, proj_bd, prev, batch, hist, hist_lo):
    tok_per_block = _KCH * _CHUNK
    blocks_per_h = batch // tok_per_block
    grid = (hist - hist_lo) * blocks_per_h
    return pl.pallas_call(
        functools.partial(_proj_body_alias),
        grid=(grid,),
        in_specs=[
            pl.BlockSpec((tok_per_block // 4, 128), lambda i: (i, 0)),
            pl.BlockSpec((_MODEL_DIM, 4 * _MODEL_DIM), lambda i: (0, 0)),
            pl.BlockSpec(memory_space=pl.ANY),
        ],
        out_specs=pl.BlockSpec(
            (1, tok_per_block, _MODEL_DIM),
            lambda i: (hist_lo + i // blocks_per_h, i % blocks_per_h, 0)),
        out_shape=jax.ShapeDtypeStruct((hist, batch, _MODEL_DIM),
                                       jnp.float32),
        input_output_aliases={2: 0},
    )(x2d, proj_bd, prev)


def kernel(token_ids, weight, proj):
    b, h = token_ids.shape
    h_lo = 20
    tok_hm = jnp.transpose(token_ids).reshape(b * h)
    tok_a = tok_hm[:h_lo * b]
    tok_b = tok_hm[h_lo * b:]
    x2d_a = _sc_gather_sum(tok_a)(tok_a, weight)
    x2d_b = _sc_gather_sum(tok_b)(tok_b, weight)
    ph = proj * 0.5
    proj_bd = jnp.zeros((_MODEL_DIM, 4 * _MODEL_DIM), jnp.float32)
    for q in range(4):
        proj_bd = proj_bd.at[q * _RANK:(q + 1) * _RANK,
                             q * _MODEL_DIM:(q + 1) * _MODEL_DIM].set(ph)
    out_a = _tc_project_first(x2d_a, proj_bd, b, h, h_lo)
    out_t = _tc_project_second(x2d_b, proj_bd, out_a, b, h, h_lo)
    return jnp.transpose(out_t, (1, 0, 2))
